# Initial kernel scaffold; baseline (speedup 1.0000x reference)
import jax
import jax.numpy as jnp
from jax.experimental import pallas as pl


def kernel(x, W):
    raise NotImplementedError("write your pallas kernel here")


# speedup vs baseline: 5.8014x; 5.8014x over previous
import jax
import jax.numpy as jnp
from jax.experimental import pallas as pl
from jax.experimental.pallas import tpu as pltpu

_N_EXPERT = 64
_EMBD_DIM = 4096
_TOP_K = 8
_TOKEN_BLOCK = 1024


def _router_block(x_ref, wt_ref, probs_ref, idx_ref):
    x = x_ref[...]
    wt = wt_ref[...]
    logits = jax.lax.dot_general(
        x, wt, (((1,), (0,)), ((), ())), preferred_element_type=jnp.float32
    )

    lane = jax.lax.broadcasted_iota(jnp.int32, logits.shape, 1)
    work = logits
    sel = jnp.zeros(logits.shape, jnp.bool_)
    idx_cols = []
    m0 = None
    for k in range(_TOP_K):
        m = jnp.max(work, axis=-1, keepdims=True)
        amax = jnp.min(
            jnp.where(work == m, lane, _N_EXPERT), axis=-1, keepdims=True
        )
        hit = lane == amax
        sel = jnp.logical_or(sel, hit)
        work = jnp.where(hit, -jnp.inf, work)
        idx_cols.append(amax)
        if k == 0:
            m0 = m

    e = jnp.where(sel, jnp.exp(logits - m0), 0.0)
    denom = jnp.sum(e, axis=-1, keepdims=True)
    probs_ref[...] = e / denom
    idx_ref[...] = jnp.concatenate(idx_cols, axis=-1)


def kernel(x, W):
    n_tokens, d = x.shape
    wt = W.T
    grid = (n_tokens // _TOKEN_BLOCK,)
    probs, idx = pl.pallas_call(
        _router_block,
        grid=grid,
        in_specs=[
            pl.BlockSpec((_TOKEN_BLOCK, d), lambda i: (i, 0)),
            pl.BlockSpec((d, _N_EXPERT), lambda i: (0, 0)),
        ],
        out_specs=[
            pl.BlockSpec((_TOKEN_BLOCK, _N_EXPERT), lambda i: (i, 0)),
            pl.BlockSpec((_TOKEN_BLOCK, _TOP_K), lambda i: (i, 0)),
        ],
        out_shape=[
            jax.ShapeDtypeStruct((n_tokens, _N_EXPERT), jnp.float32),
            jax.ShapeDtypeStruct((n_tokens, _TOP_K), jnp.int32),
        ],
        compiler_params=pltpu.CompilerParams(
            dimension_semantics=("parallel",)
        ),
    )(x, wt)
    return (probs, idx)


# speedup vs baseline: 6.6827x; 1.1519x over previous
import jax
import jax.numpy as jnp
from jax.experimental import pallas as pl
from jax.experimental.pallas import tpu as pltpu

_N_EXPERT = 64
_EMBD_DIM = 4096
_TOP_K = 8
_TOKEN_BLOCK = 1024


_SUB = 256


def _topk_softmax(logits):
    lt = logits.T
    expert = jax.lax.broadcasted_iota(jnp.int32, lt.shape, 0)
    work = lt
    sel = jnp.zeros(lt.shape, jnp.bool_)
    idx_rows = []
    m0 = None
    for k in range(_TOP_K):
        m = jnp.max(work, axis=0, keepdims=True)
        amax = jnp.min(
            jnp.where(work == m, expert, _N_EXPERT), axis=0, keepdims=True
        )
        hit = expert == amax
        sel = jnp.logical_or(sel, hit)
        work = jnp.where(hit, -jnp.inf, work)
        idx_rows.append(amax)
        if k == 0:
            m0 = m

    e = jnp.where(sel, jnp.exp(lt - m0), 0.0)
    denom = jnp.sum(e, axis=0, keepdims=True)
    probs_t = e / denom
    idx_t = jnp.concatenate(idx_rows, axis=0)
    return probs_t.T, idx_t.T


def _router_block(x_ref, wt_ref, probs_ref, idx_ref):
    x = x_ref[...]
    wt = wt_ref[...]
    logits = jax.lax.dot_general(
        x, wt, (((1,), (0,)), ((), ())), preferred_element_type=jnp.float32
    )
    for s in range(_TOKEN_BLOCK // _SUB):
        lo, hi = s * _SUB, (s + 1) * _SUB
        probs, idx = _topk_softmax(logits[lo:hi, :])
        probs_ref[lo:hi, :] = probs
        idx_ref[lo:hi, :] = idx


def kernel(x, W):
    n_tokens, d = x.shape
    wt = W.T
    grid = (n_tokens // _TOKEN_BLOCK,)
    probs, idx = pl.pallas_call(
        _router_block,
        grid=grid,
        in_specs=[
            pl.BlockSpec((_TOKEN_BLOCK, d), lambda i: (i, 0)),
            pl.BlockSpec((d, _N_EXPERT), lambda i: (0, 0)),
        ],
        out_specs=[
            pl.BlockSpec((_TOKEN_BLOCK, _N_EXPERT), lambda i: (i, 0)),
            pl.BlockSpec((_TOKEN_BLOCK, _TOP_K), lambda i: (i, 0)),
        ],
        out_shape=[
            jax.ShapeDtypeStruct((n_tokens, _N_EXPERT), jnp.float32),
            jax.ShapeDtypeStruct((n_tokens, _TOP_K), jnp.int32),
        ],
        compiler_params=pltpu.CompilerParams(
            dimension_semantics=("parallel",)
        ),
    )(x, wt)
    return (probs, idx)
